# baseline (speedup 1.0000x reference)
import functools

import jax
import jax.numpy as jnp
from jax import lax
from jax.experimental import pallas as pl
from jax.experimental.pallas import tpu as pltpu
from jax.experimental.pallas import tpu_sc as plsc

NC = 2
NS = 16
NW = NC * NS


@functools.lru_cache(maxsize=None)
def _build(B, D):
    b_per_w = B // NW
    mesh = plsc.VectorSubcoreMesh(core_axis_name="c", subcore_axis_name="s")

    @functools.partial(
        pl.kernel,
        mesh=mesh,
        out_type=(
            jax.ShapeDtypeStruct((B, D), jnp.float32),
            jax.ShapeDtypeStruct((B, D), jnp.float32),
        ),
        scratch_types=[
            pltpu.VMEM((b_per_w,), jnp.int32),
            pltpu.VMEM((b_per_w,), jnp.int32),
            pltpu.SemaphoreType.DMA,
            pltpu.SemaphoreType.DMA,
        ],
        compiler_params=pltpu.CompilerParams(
            use_tc_tiling_on_sc=True, needs_layout_passes=False),
    )
    def k(ori_hbm, dest_hbm, table_hbm, out_o_hbm, out_d_hbm,
          idx_o, idx_d, sem_o, sem_d):
        wid = lax.axis_index("s") * NC + lax.axis_index("c")
        base = wid * b_per_w
        pltpu.sync_copy(ori_hbm.at[pl.ds(base, b_per_w)], idx_o)
        pltpu.sync_copy(dest_hbm.at[pl.ds(base, b_per_w)], idx_d)

        def fire(idx, out_hbm, sem):
            def body(g, c):
                v = idx[pl.ds(g * 16, 16)]
                for kk in range(16):
                    pltpu.make_async_copy(
                        table_hbm.at[pl.ds(v[kk], 1)],
                        out_hbm.at[pl.ds(base + g * 16 + kk, 1)],
                        sem).start()
                return c
            lax.fori_loop(0, b_per_w // 16, body, 0)

        fire(idx_o, out_o_hbm, sem_o)
        fire(idx_d, out_d_hbm, sem_d)
        pltpu.make_async_copy(
            table_hbm.at[pl.ds(0, b_per_w)],
            out_o_hbm.at[pl.ds(base, b_per_w)], sem_o).wait()
        pltpu.make_async_copy(
            table_hbm.at[pl.ds(0, b_per_w)],
            out_d_hbm.at[pl.ds(base, b_per_w)], sem_d).wait()

    return k


def kernel(ori, dest, table):
    B = ori.shape[0]
    D = table.shape[1]
    return _build(B, D)(ori.astype(jnp.int32), dest.astype(jnp.int32), table)


# speedup vs baseline: 2.2881x; 2.2881x over previous
import functools

import jax
import jax.numpy as jnp
from jax import lax
from jax.experimental import pallas as pl
from jax.experimental.pallas import tpu as pltpu
from jax.experimental.pallas import tpu_sc as plsc

NC = 2
NS = 16
NW = NC * NS


@functools.lru_cache(maxsize=None)
def _build(B, D):
    b_per_w = B // NW
    mesh = plsc.VectorSubcoreMesh(core_axis_name="c", subcore_axis_name="s")

    @functools.partial(
        pl.kernel,
        mesh=mesh,
        out_type=(
            jax.ShapeDtypeStruct((B, D), jnp.float32),
            jax.ShapeDtypeStruct((B, D), jnp.float32),
        ),
        scratch_types=[
            pltpu.VMEM((b_per_w,), jnp.int32),
            pltpu.VMEM((b_per_w,), jnp.int32),
            pltpu.VMEM((b_per_w, D), jnp.float32),
            pltpu.SemaphoreType.DMA,
        ],
        compiler_params=pltpu.CompilerParams(
            use_tc_tiling_on_sc=True, needs_layout_passes=False),
    )
    def k(ori_hbm, dest_hbm, table_hbm, out_o_hbm, out_d_hbm,
          idx_o, idx_d, rows, sem):
        wid = lax.axis_index("s") * NC + lax.axis_index("c")
        base = wid * b_per_w
        pltpu.sync_copy(ori_hbm.at[pl.ds(base, b_per_w)], idx_o)
        pltpu.sync_copy(dest_hbm.at[pl.ds(base, b_per_w)], idx_d)

        def run(idx, out_hbm):
            def body(g, c):
                v = idx[pl.ds(g * 16, 16)]
                for kk in range(16):
                    pltpu.make_async_copy(
                        table_hbm.at[pl.ds(v[kk], 1)],
                        rows.at[pl.ds(g * 16 + kk, 1)], sem).start()
                return c
            lax.fori_loop(0, b_per_w // 16, body, 0)
            pltpu.make_async_copy(
                table_hbm.at[pl.ds(0, b_per_w)], rows, sem).wait()
            pltpu.sync_copy(rows, out_hbm.at[pl.ds(base, b_per_w)])

        run(idx_o, out_o_hbm)
        run(idx_d, out_d_hbm)

    return k


def kernel(ori, dest, table):
    B = ori.shape[0]
    D = table.shape[1]
    return _build(B, D)(ori.astype(jnp.int32), dest.astype(jnp.int32), table)


# speedup vs baseline: 2.2926x; 1.0020x over previous
import functools

import jax
import jax.numpy as jnp
from jax import lax
from jax.experimental import pallas as pl
from jax.experimental.pallas import tpu as pltpu
from jax.experimental.pallas import tpu_sc as plsc

NC = 2
NS = 16
NW = NC * NS


@functools.lru_cache(maxsize=None)
def _build(B, D):
    b_per_w = B // NW
    mesh = plsc.VectorSubcoreMesh(core_axis_name="c", subcore_axis_name="s")

    @functools.partial(
        pl.kernel,
        mesh=mesh,
        out_type=(
            jax.ShapeDtypeStruct((B, D), jnp.float32),
            jax.ShapeDtypeStruct((B, D), jnp.float32),
        ),
        scratch_types=[
            pltpu.VMEM((b_per_w,), jnp.int32),
            pltpu.VMEM((b_per_w,), jnp.int32),
            pltpu.VMEM((b_per_w, D), jnp.float32),
            pltpu.SemaphoreType.DMA,
        ],
        compiler_params=pltpu.CompilerParams(
            use_tc_tiling_on_sc=True, needs_layout_passes=False),
    )
    def k(ori_hbm, dest_hbm, table_hbm, out_o_hbm, out_d_hbm,
          idx_o, idx_d, rows, sem):
        wid = lax.axis_index("s") * NC + lax.axis_index("c")
        base = wid * b_per_w
        pltpu.sync_copy(ori_hbm.at[pl.ds(base, b_per_w)], idx_o)
        pltpu.sync_copy(dest_hbm.at[pl.ds(base, b_per_w)], idx_d)

        def run(idx, out_hbm):
            def body(g, c):
                for h in range(4):
                    v = idx[pl.ds(g * 64 + h * 16, 16)]
                    for kk in range(16):
                        pltpu.make_async_copy(
                            table_hbm.at[pl.ds(v[kk], 1)],
                            rows.at[pl.ds(g * 64 + h * 16 + kk, 1)],
                            sem).start()
                return c
            lax.fori_loop(0, b_per_w // 64, body, 0)
            pltpu.make_async_copy(
                table_hbm.at[pl.ds(0, b_per_w)], rows, sem).wait()
            pltpu.sync_copy(rows, out_hbm.at[pl.ds(base, b_per_w)])

        run(idx_o, out_o_hbm)
        run(idx_d, out_d_hbm)

    return k


def kernel(ori, dest, table):
    B = ori.shape[0]
    D = table.shape[1]
    return _build(B, D)(ori.astype(jnp.int32), dest.astype(jnp.int32), table)
